# baseline (speedup 1.0000x reference)
import jax
import jax.numpy as jnp
from jax import lax
from jax.experimental import pallas as pl
from jax.experimental.pallas import tpu as pltpu
from jax.experimental.pallas import tpu_sc as plsc

BATCH = 4096
HIST = 200
D_MODEL = 128
NUM_INDICES = BATCH * HIST
NC, NS = 2, 16
NW = NC * NS
N_PER_W = NUM_INDICES // NW
CHUNK = 200
NBUF = 4
N_CHUNKS = N_PER_W // CHUNK


def kernel(timesteps, pe):
    indices = timesteps.reshape((NUM_INDICES,))

    vector_mesh = plsc.VectorSubcoreMesh(
        core_axis_name="core", subcore_axis_name="subcore"
    )

    @jax.jit
    def gather(pe, indices):
        @pl.kernel(
            out_type=jax.ShapeDtypeStruct((NUM_INDICES, D_MODEL), pe.dtype),
            mesh=vector_mesh,
            scratch_types=[
                pltpu.VMEM((N_PER_W,), jnp.int32),
                pltpu.VMEM((NBUF, CHUNK, D_MODEL), jnp.float32),
                pltpu.SemaphoreType.DMA,
                pltpu.SemaphoreType.DMA((NBUF,)),
                pltpu.SemaphoreType.DMA((NBUF,)),
            ],
        )
        def sc_kernel(pe_hbm, i_hbm, o_hbm, idx_v, rows_v, isem, gsem, osem):
            wid = lax.axis_index("subcore") * NC + lax.axis_index("core")
            base = wid * N_PER_W
            pltpu.async_copy(i_hbm.at[pl.ds(base, N_PER_W)], idx_v, isem).wait()

            def start_gather(k, b):
                pltpu.async_copy(
                    pe_hbm.at[idx_v.at[pl.ds(k * CHUNK, CHUNK)]],
                    rows_v.at[b],
                    gsem.at[b],
                )

            def wait_gather(b):
                pltpu.make_async_copy(
                    pe_hbm.at[idx_v.at[pl.ds(0, CHUNK)]],
                    rows_v.at[b],
                    gsem.at[b],
                ).wait()

            def start_out(k, b):
                pltpu.async_copy(
                    rows_v.at[b],
                    o_hbm.at[pl.ds(base + k * CHUNK, CHUNK)],
                    osem.at[b],
                )

            def wait_out(b):
                pltpu.make_async_copy(
                    rows_v.at[b],
                    o_hbm.at[pl.ds(base, CHUNK)],
                    osem.at[b],
                ).wait()

            for b in range(NBUF):
                start_gather(b, b)

            @pl.loop(0, N_CHUNKS, step=NBUF)
            def _(g):
                for b in range(NBUF):
                    k = g + b
                    wait_gather(b)
                    start_out(k, b)

                    @pl.when(k + NBUF < N_CHUNKS)
                    def _():
                        wait_out(b)
                        start_gather(k + NBUF, b)

            for b in range(NBUF):
                wait_out(b)

        return sc_kernel(pe, indices)

    out = gather(pe, indices)
    return out.reshape((BATCH, HIST, D_MODEL))
